# baseline (speedup 1.0000x reference)
import dataclasses
import functools

import numpy as np

import jax
import jax.numpy as jnp
from jax import lax
from jax.experimental import pallas as pl
from jax.experimental.pallas import tpu as pltpu
from jax.experimental.pallas import tpu_sc as plsc

N = 10000
E = 320000
D = 128
N_PAD = 10240
NC = 2
NS = 16
NW = NC * NS
CHUNK = 128
NCHUNKS_TOT = E // CHUNK
NCH_BASE = NCHUNKS_TOT // NW
NCH_REM = NCHUNKS_TOT - NCH_BASE * NW
NCHUNK = NCH_BASE + 1
ROWS_PER_SUB = N_PAD // NS
LSUB = 16

_mesh = plsc.VectorSubcoreMesh(core_axis_name="c", subcore_axis_name="s")

_sc_params = pltpu.CompilerParams()
if "needs_layout_passes" in pltpu.CompilerParams.__dataclass_fields__:
    _sc_params = dataclasses.replace(_sc_params, needs_layout_passes=False)
if "use_tc_tiling_on_sc" in pltpu.CompilerParams.__dataclass_fields__:
    _sc_params = dataclasses.replace(_sc_params, use_tc_tiling_on_sc=False)


def _worker_chunks(wid):
    cbase = wid * NCH_BASE + jnp.minimum(wid, NCH_REM)
    extra = wid < NCH_REM
    srow = jnp.where(extra, cbase + NCH_BASE, cbase)
    return cbase, extra, srow


def _sc_deg(ei3, ew2d):
    @functools.partial(
        pl.kernel,
        out_type=jax.ShapeDtypeStruct((NC, N_PAD, LSUB), jnp.float32),
        mesh=_mesh,
        scratch_types=[
            pltpu.VMEM((NCHUNK, CHUNK), jnp.int32),
            pltpu.VMEM((NCHUNK, CHUNK), jnp.float32),
            pltpu.VMEM((CHUNK, LSUB), jnp.float32),
            pltpu.VMEM((CHUNK, LSUB), jnp.float32),
            pltpu.VMEM_SHARED((N_PAD, LSUB), jnp.float32),
            pltpu.SemaphoreType.DMA,
            pltpu.SemaphoreType.DMA,
        ],
        compiler_params=_sc_params,
    )
    def k(ei_hbm, ew_hbm, out_hbm, cidx, ewb, msg0, msg1, acc, s0, s1):
        c = lax.axis_index("c")
        s = lax.axis_index("s")
        wid = s * NC + c
        cbase, extra, srow = _worker_chunks(wid)
        col_hbm = ei_hbm.at[1]
        l0 = pltpu.async_copy(col_hbm.at[pl.ds(cbase, NCH_BASE)],
                              cidx.at[pl.ds(0, NCH_BASE)], s0)
        l1 = pltpu.async_copy(col_hbm.at[pl.ds(srow, 1)],
                              cidx.at[pl.ds(NCH_BASE, 1)], s0)
        l2 = pltpu.async_copy(ew_hbm.at[pl.ds(cbase, NCH_BASE)],
                              ewb.at[pl.ds(0, NCH_BASE)], s0)
        l3 = pltpu.async_copy(ew_hbm.at[pl.ds(srow, 1)],
                              ewb.at[pl.ds(NCH_BASE, 1)], s0)
        zero = jnp.zeros((LSUB,), jnp.float32)

        @pl.loop(0, CHUNK)
        def _(e):
            msg0[e, :] = zero

        @pl.loop(0, ROWS_PER_SUB // CHUNK)
        def _(i):
            pltpu.sync_copy(msg0, acc.at[pl.ds(s * ROWS_PER_SUB + i * CHUNK, CHUNK)])

        l0.wait()
        l1.wait()
        l2.wait()
        l3.wait()
        plsc.subcore_barrier()

        def build(kk, msg):
            @plsc.parallel_loop(0, CHUNK // LSUB, 1, unroll=CHUNK // LSUB)
            def _(g):
                ew16 = ewb[kk, pl.ds(g * LSUB, LSUB)]
                for l in range(LSUB):
                    msg[g * LSUB + l, :] = jnp.full((LSUB,), ew16[l],
                                                    jnp.float32)

        @pl.loop(0, (NCHUNK - 1) // 2)
        def _(i):
            a = 2 * i
            build(a, msg0)
            d0 = pltpu.async_copy(msg0, acc.at[cidx.at[a]], s0, add=True)
            build(a + 1, msg1)
            d1 = pltpu.async_copy(msg1, acc.at[cidx.at[a + 1]], s1, add=True)
            d0.wait()
            d1.wait()

        @pl.when(extra)
        def _():
            build(NCHUNK - 1, msg0)
            pltpu.sync_copy(msg0, acc.at[cidx.at[NCHUNK - 1]], add=True)

        plsc.subcore_barrier()
        pltpu.sync_copy(
            acc.at[pl.ds(s * ROWS_PER_SUB, ROWS_PER_SUB)],
            out_hbm.at[c].at[pl.ds(s * ROWS_PER_SUB, ROWS_PER_SUB)],
        )

    return k(ei3, ew2d)


def _sc_agg(xw, dis, ei3, ew2d, F):
    @functools.partial(
        pl.kernel,
        out_type=jax.ShapeDtypeStruct((NC, N_PAD, F), jnp.float32),
        mesh=_mesh,
        scratch_types=[
            pltpu.VMEM((N_PAD,), jnp.float32),
            pltpu.VMEM((NCHUNK, CHUNK), jnp.int32),
            pltpu.VMEM((NCHUNK, CHUNK), jnp.int32),
            pltpu.VMEM((NCHUNK, CHUNK), jnp.float32),
            pltpu.VMEM((NCHUNK, CHUNK), jnp.float32),
            pltpu.VMEM((CHUNK, F), jnp.float32),
            pltpu.VMEM((CHUNK, F), jnp.float32),
            pltpu.VMEM((CHUNK, F), jnp.float32),
            pltpu.VMEM((CHUNK, F), jnp.float32),
            pltpu.VMEM_SHARED((N_PAD, F), jnp.float32),
            pltpu.SemaphoreType.DMA,
            pltpu.SemaphoreType.DMA,
            pltpu.SemaphoreType.DMA,
            pltpu.SemaphoreType.DMA,
        ],
        compiler_params=_sc_params,
    )
    def k(xw_hbm, dis_hbm, ei_hbm, ew_hbm, out_hbm,
          disv, ridx, cidx, ewb, nrm, gbuf0, gbuf1, sbuf0, sbuf1,
          acc, s0, s1, ss0, ss1):
        c = lax.axis_index("c")
        s = lax.axis_index("s")
        wid = s * NC + c
        cbase, extra, srow = _worker_chunks(wid)
        row_hbm = ei_hbm.at[0]
        col_hbm = ei_hbm.at[1]
        ls = [pltpu.async_copy(dis_hbm, disv, s0)]
        for src, dst in ((row_hbm, ridx), (col_hbm, cidx), (ew_hbm, ewb)):
            ls.append(pltpu.async_copy(src.at[pl.ds(cbase, NCH_BASE)],
                                       dst.at[pl.ds(0, NCH_BASE)], s0))
            ls.append(pltpu.async_copy(src.at[pl.ds(srow, 1)],
                                       dst.at[pl.ds(NCH_BASE, 1)], s0))
        zero = jnp.zeros((LSUB,), jnp.float32)

        @pl.loop(0, CHUNK)
        def _(e):
            for f in range(F // LSUB):
                sbuf0[e, pl.ds(f * LSUB, LSUB)] = zero

        @pl.loop(0, ROWS_PER_SUB // CHUNK)
        def _(i):
            pltpu.sync_copy(sbuf0, acc.at[pl.ds(s * ROWS_PER_SUB + i * CHUNK, CHUNK)])

        for l in ls:
            l.wait()

        g0 = pltpu.async_copy(xw_hbm.at[ridx.at[0]], gbuf0, s0)

        @plsc.parallel_loop(0, NCHUNK)
        def _(kk):
            @plsc.parallel_loop(0, CHUNK // LSUB, 1, unroll=CHUNK // LSUB)
            def _(g):
                sl = pl.ds(g * LSUB, LSUB)
                dr = plsc.load_gather(disv, [ridx[kk, sl]])
                dc = plsc.load_gather(disv, [cidx[kk, sl]])
                nrm[kk, sl] = dr * ewb[kk, sl] * dc

        plsc.subcore_barrier()

        def scale(kk, gbuf, sbuf):
            @plsc.parallel_loop(0, CHUNK // LSUB, 1, unroll=CHUNK // LSUB)
            def _(g):
                n16 = nrm[kk, pl.ds(g * LSUB, LSUB)]
                for l in range(LSUB):
                    w = n16[l]
                    for f in range(F // LSUB):
                        sl = pl.ds(f * LSUB, LSUB)
                        sbuf[g * LSUB + l, sl] = gbuf[g * LSUB + l, sl] * w

        g0.wait()

        @pl.loop(0, (NCHUNK - 1) // 2)
        def _(i):
            a = 2 * i
            d1 = pltpu.async_copy(xw_hbm.at[ridx.at[a + 1]], gbuf1, s1)
            scale(a, gbuf0, sbuf0)
            sc0 = pltpu.async_copy(sbuf0, acc.at[cidx.at[a]], ss0, add=True)
            d0 = pltpu.async_copy(xw_hbm.at[ridx.at[a + 2]], gbuf0, s0)
            d1.wait()
            scale(a + 1, gbuf1, sbuf1)
            sc1 = pltpu.async_copy(sbuf1, acc.at[cidx.at[a + 1]], ss1, add=True)
            sc0.wait()
            sc1.wait()
            d0.wait()

        @pl.when(extra)
        def _():
            scale(NCHUNK - 1, gbuf0, sbuf0)
            pltpu.sync_copy(sbuf0, acc.at[cidx.at[NCHUNK - 1]], add=True)

        plsc.subcore_barrier()
        pltpu.sync_copy(
            acc.at[pl.ds(s * ROWS_PER_SUB, ROWS_PER_SUB)],
            out_hbm.at[c].at[pl.ds(s * ROWS_PER_SUB, ROWS_PER_SUB)],
        )

    return k(xw, dis, ei3, ew2d)


def _tc1(degp, x, W1):
    def body(degp_ref, x_ref, w1_ref, dis_ref, xw_ref):
        deg = degp_ref[0][:, 0:1] + degp_ref[1][:, 0:1] + 1.0
        dis_ref[...] = lax.rsqrt(deg)
        xw_ref[...] = jnp.dot(x_ref[...], w1_ref[...],
                              preferred_element_type=jnp.float32)

    nb = 8
    blk = N_PAD // nb
    return pl.pallas_call(
        body,
        grid=(nb,),
        in_specs=[
            pl.BlockSpec((NC, blk, LSUB), lambda i: (0, i, 0)),
            pl.BlockSpec((blk, D), lambda i: (i, 0)),
            pl.BlockSpec((D, 32), lambda i: (0, 0)),
        ],
        out_specs=[
            pl.BlockSpec((blk, 1), lambda i: (i, 0)),
            pl.BlockSpec((blk, 32), lambda i: (i, 0)),
        ],
        out_shape=[
            jax.ShapeDtypeStruct((N_PAD, 1), jnp.float32),
            jax.ShapeDtypeStruct((N_PAD, 32), jnp.float32),
        ],
    )(degp, x, W1)


def _tc2(agg, dis, xw, b1, W2):
    def body(a_ref, dis_ref, xw_ref, b1_ref, w2_ref, hw2_ref):
        dis = dis_ref[...]
        h = a_ref[0] + a_ref[1] + dis * dis * xw_ref[...] + b1_ref[...]
        h = jnp.maximum(h, 0.0)
        hw2_ref[...] = jnp.dot(h, w2_ref[...],
                               preferred_element_type=jnp.float32)

    nb = 8
    blk = N_PAD // nb
    return pl.pallas_call(
        body,
        grid=(nb,),
        in_specs=[
            pl.BlockSpec((NC, blk, 32), lambda i: (0, i, 0)),
            pl.BlockSpec((blk, 1), lambda i: (i, 0)),
            pl.BlockSpec((blk, 32), lambda i: (i, 0)),
            pl.BlockSpec((1, 32), lambda i: (0, 0)),
            pl.BlockSpec((32, 16), lambda i: (0, 0)),
        ],
        out_specs=pl.BlockSpec((blk, 16), lambda i: (i, 0)),
        out_shape=jax.ShapeDtypeStruct((N_PAD, 16), jnp.float32),
    )(agg, dis, xw, b1, W2)


def _tc3(agg, dis, hw2, b2, W_out, b_out):
    def body(q_ref, dis_ref, hw2_ref, b2_ref, wo_ref, bo_ref, out_ref):
        dis = dis_ref[...]
        h = q_ref[0] + q_ref[1] + dis * dis * hw2_ref[...] + b2_ref[...]
        h = jnp.maximum(h, 0.0)
        out_ref[...] = jnp.dot(h, wo_ref[...],
                               preferred_element_type=jnp.float32) + bo_ref[...]

    nb = 8
    blk = N_PAD // nb
    return pl.pallas_call(
        body,
        grid=(nb,),
        in_specs=[
            pl.BlockSpec((NC, blk, 16), lambda i: (0, i, 0)),
            pl.BlockSpec((blk, 1), lambda i: (i, 0)),
            pl.BlockSpec((blk, 16), lambda i: (i, 0)),
            pl.BlockSpec((1, 16), lambda i: (0, 0)),
            pl.BlockSpec((16, 3), lambda i: (0, 0)),
            pl.BlockSpec((1, 3), lambda i: (0, 0)),
        ],
        out_specs=pl.BlockSpec((blk, 3), lambda i: (i, 0)),
        out_shape=jax.ShapeDtypeStruct((N_PAD, 3), jnp.float32),
    )(agg, dis, hw2, b2, W_out, b_out)


def kernel(x, edge_index, edge_weight, W1, b1, W2, b2, W_out, b_out):
    ei3 = edge_index.astype(jnp.int32).reshape(2, NCHUNKS_TOT, CHUNK)
    ew2d = edge_weight.astype(jnp.float32).reshape(NCHUNKS_TOT, CHUNK)
    x_pad = jnp.concatenate(
        [x, jnp.zeros((N_PAD - N, D), jnp.float32)], axis=0)

    degp = _sc_deg(ei3, ew2d)
    dis2d, xw = _tc1(degp, x_pad, W1)
    dis = dis2d.reshape(N_PAD)

    a1 = _sc_agg(xw, dis, ei3, ew2d, 32)
    hw2 = _tc2(a1, dis2d, xw, b1.reshape(1, 32), W2)

    a2 = _sc_agg(hw2, dis, ei3, ew2d, 16)
    out = _tc3(a2, dis2d, hw2, b2.reshape(1, 16), W_out, b_out.reshape(1, 3))
    return out[:N]


# speedup vs baseline: 1.0160x; 1.0160x over previous
import dataclasses
import functools

import jax
import jax.numpy as jnp
from jax import lax
from jax.experimental import pallas as pl
from jax.experimental.pallas import tpu as pltpu
from jax.experimental.pallas import tpu_sc as plsc

N = 10000
E = 320000
D = 128
N_PAD = 10240
NC = 2
NS = 16
NW = NC * NS
CHUNK = 128
NCHUNKS_TOT = E // CHUNK
NCH_BASE = NCHUNKS_TOT // NW
NCH_REM = NCHUNKS_TOT - NCH_BASE * NW
NCHUNK = NCH_BASE + 1
ROWS_PER_SUB = N_PAD // NS
LSUB = 16

_mesh = plsc.VectorSubcoreMesh(core_axis_name="c", subcore_axis_name="s")

_sc_params = pltpu.CompilerParams()
if "needs_layout_passes" in pltpu.CompilerParams.__dataclass_fields__:
    _sc_params = dataclasses.replace(_sc_params, needs_layout_passes=False)
if "use_tc_tiling_on_sc" in pltpu.CompilerParams.__dataclass_fields__:
    _sc_params = dataclasses.replace(_sc_params, use_tc_tiling_on_sc=False)


def _worker_chunks(wid):
    cbase = wid * NCH_BASE + jnp.minimum(wid, NCH_REM)
    extra = wid < NCH_REM
    srow = jnp.where(extra, cbase + NCH_BASE, cbase)
    return cbase, extra, srow


def _sc_deg(ei3, ew2d):
    @functools.partial(
        pl.kernel,
        out_type=jax.ShapeDtypeStruct((NC, N_PAD, LSUB), jnp.float32),
        mesh=_mesh,
        scratch_types=[
            pltpu.VMEM((NCHUNK, CHUNK), jnp.int32),
            pltpu.VMEM((NCHUNK, CHUNK), jnp.float32),
            pltpu.VMEM((CHUNK, LSUB), jnp.float32),
            pltpu.VMEM((CHUNK, LSUB), jnp.float32),
            pltpu.VMEM_SHARED((N_PAD, LSUB), jnp.float32),
            pltpu.SemaphoreType.DMA,
            pltpu.SemaphoreType.DMA,
        ],
        compiler_params=_sc_params,
    )
    def k(ei_hbm, ew_hbm, out_hbm, cidx, ewb, msg0, msg1, acc, s0, s1):
        c = lax.axis_index("c")
        s = lax.axis_index("s")
        wid = s * NC + c
        cbase, extra, srow = _worker_chunks(wid)
        col_hbm = ei_hbm.at[1]
        l0 = pltpu.async_copy(col_hbm.at[pl.ds(cbase, NCH_BASE)],
                              cidx.at[pl.ds(0, NCH_BASE)], s0)
        l1 = pltpu.async_copy(col_hbm.at[pl.ds(srow, 1)],
                              cidx.at[pl.ds(NCH_BASE, 1)], s0)
        l2 = pltpu.async_copy(ew_hbm.at[pl.ds(cbase, NCH_BASE)],
                              ewb.at[pl.ds(0, NCH_BASE)], s0)
        l3 = pltpu.async_copy(ew_hbm.at[pl.ds(srow, 1)],
                              ewb.at[pl.ds(NCH_BASE, 1)], s0)
        zero = jnp.zeros((LSUB,), jnp.float32)

        @pl.loop(0, CHUNK)
        def _(e):
            msg0[e, :] = zero

        @pl.loop(0, ROWS_PER_SUB // CHUNK)
        def _(i):
            pltpu.sync_copy(msg0, acc.at[pl.ds(s * ROWS_PER_SUB + i * CHUNK, CHUNK)])

        l0.wait()
        l1.wait()
        l2.wait()
        l3.wait()
        plsc.subcore_barrier()

        def build(kk, msg):
            @plsc.parallel_loop(0, CHUNK // LSUB, 1, unroll=CHUNK // LSUB)
            def _(g):
                ew16 = ewb[kk, pl.ds(g * LSUB, LSUB)]
                for l in range(LSUB):
                    msg[g * LSUB + l, :] = jnp.full((LSUB,), ew16[l],
                                                    jnp.float32)

        @pl.loop(0, (NCHUNK - 1) // 2)
        def _(i):
            a = 2 * i
            build(a, msg0)
            d0 = pltpu.async_copy(msg0, acc.at[cidx.at[a]], s0, add=True)
            build(a + 1, msg1)
            d1 = pltpu.async_copy(msg1, acc.at[cidx.at[a + 1]], s1, add=True)
            d0.wait()
            d1.wait()

        @pl.when(extra)
        def _():
            build(NCHUNK - 1, msg0)
            pltpu.sync_copy(msg0, acc.at[cidx.at[NCHUNK - 1]], add=True)

        plsc.subcore_barrier()
        pltpu.sync_copy(
            acc.at[pl.ds(s * ROWS_PER_SUB, ROWS_PER_SUB)],
            out_hbm.at[c].at[pl.ds(s * ROWS_PER_SUB, ROWS_PER_SUB)],
        )

    return k(ei3, ew2d)


def _sc_agg(xw, dis, ei3, ew2d, F):
    @functools.partial(
        pl.kernel,
        out_type=jax.ShapeDtypeStruct((NC, N_PAD, F), jnp.float32),
        mesh=_mesh,
        scratch_types=[
            pltpu.VMEM((N_PAD,), jnp.float32),
            pltpu.VMEM((NCHUNK, CHUNK), jnp.int32),
            pltpu.VMEM((NCHUNK, CHUNK), jnp.int32),
            pltpu.VMEM((NCHUNK, CHUNK), jnp.float32),
            pltpu.VMEM((NCHUNK, CHUNK), jnp.float32),
            pltpu.VMEM((CHUNK, F), jnp.float32),
            pltpu.VMEM((CHUNK, F), jnp.float32),
            pltpu.VMEM((CHUNK, F), jnp.float32),
            pltpu.VMEM((CHUNK, F), jnp.float32),
            pltpu.VMEM_SHARED((N_PAD, F), jnp.float32),
            pltpu.SemaphoreType.DMA,
            pltpu.SemaphoreType.DMA,
            pltpu.SemaphoreType.DMA,
            pltpu.SemaphoreType.DMA,
        ],
        compiler_params=_sc_params,
    )
    def k(xw_hbm, dis_hbm, ei_hbm, ew_hbm, out_hbm,
          disv, ridx, cidx, ewb, nrm, gbuf0, gbuf1, sbuf0, sbuf1,
          acc, s0, s1, ss0, ss1):
        c = lax.axis_index("c")
        s = lax.axis_index("s")
        wid = s * NC + c
        cbase, extra, srow = _worker_chunks(wid)
        row_hbm = ei_hbm.at[0]
        col_hbm = ei_hbm.at[1]
        ls = [pltpu.async_copy(dis_hbm, disv, s0)]
        for src, dst in ((row_hbm, ridx), (col_hbm, cidx), (ew_hbm, ewb)):
            ls.append(pltpu.async_copy(src.at[pl.ds(cbase, NCH_BASE)],
                                       dst.at[pl.ds(0, NCH_BASE)], s0))
            ls.append(pltpu.async_copy(src.at[pl.ds(srow, 1)],
                                       dst.at[pl.ds(NCH_BASE, 1)], s0))
        zero = jnp.zeros((LSUB,), jnp.float32)

        @pl.loop(0, CHUNK)
        def _(e):
            for f in range(F // LSUB):
                sbuf0[e, pl.ds(f * LSUB, LSUB)] = zero

        @pl.loop(0, ROWS_PER_SUB // CHUNK)
        def _(i):
            pltpu.sync_copy(sbuf0, acc.at[pl.ds(s * ROWS_PER_SUB + i * CHUNK, CHUNK)])

        for l in ls:
            l.wait()

        g0 = pltpu.async_copy(xw_hbm.at[ridx.at[0]], gbuf0, s0)

        @plsc.parallel_loop(0, NCHUNK)
        def _(kk):
            @plsc.parallel_loop(0, CHUNK // LSUB, 1, unroll=CHUNK // LSUB)
            def _(g):
                sl = pl.ds(g * LSUB, LSUB)
                dr = plsc.load_gather(disv, [ridx[kk, sl]])
                dc = plsc.load_gather(disv, [cidx[kk, sl]])
                nrm[kk, sl] = dr * ewb[kk, sl] * dc

        plsc.subcore_barrier()

        def scale(kk, gbuf, sbuf):
            @plsc.parallel_loop(0, CHUNK // LSUB, 1, unroll=CHUNK // LSUB)
            def _(g):
                n16 = nrm[kk, pl.ds(g * LSUB, LSUB)]
                for l in range(LSUB):
                    w = n16[l]
                    for f in range(F // LSUB):
                        sl = pl.ds(f * LSUB, LSUB)
                        sbuf[g * LSUB + l, sl] = gbuf[g * LSUB + l, sl] * w

        g0.wait()

        @pl.loop(0, (NCHUNK - 1) // 2)
        def _(i):
            a = 2 * i
            d1 = pltpu.async_copy(xw_hbm.at[ridx.at[a + 1]], gbuf1, s1)
            scale(a, gbuf0, sbuf0)
            sc0 = pltpu.async_copy(sbuf0, acc.at[cidx.at[a]], ss0, add=True)
            d0 = pltpu.async_copy(xw_hbm.at[ridx.at[a + 2]], gbuf0, s0)
            d1.wait()
            scale(a + 1, gbuf1, sbuf1)
            sc1 = pltpu.async_copy(sbuf1, acc.at[cidx.at[a + 1]], ss1, add=True)
            sc0.wait()
            sc1.wait()
            d0.wait()

        @pl.when(extra)
        def _():
            scale(NCHUNK - 1, gbuf0, sbuf0)
            pltpu.sync_copy(sbuf0, acc.at[cidx.at[NCHUNK - 1]], add=True)

        plsc.subcore_barrier()
        pltpu.sync_copy(
            acc.at[pl.ds(s * ROWS_PER_SUB, ROWS_PER_SUB)],
            out_hbm.at[c].at[pl.ds(s * ROWS_PER_SUB, ROWS_PER_SUB)],
        )

    return k(xw, dis, ei3, ew2d)


def _tc1(degp, x, W1):
    def body(degp_ref, x_ref, w1_ref, dis_ref, xw_ref):
        deg = degp_ref[0][:, 0:1] + degp_ref[1][:, 0:1] + 1.0
        dis_ref[...] = lax.rsqrt(deg)
        xw_ref[...] = jnp.dot(x_ref[...], w1_ref[...],
                              preferred_element_type=jnp.float32)

    return pl.pallas_call(
        body,
        out_shape=[
            jax.ShapeDtypeStruct((N_PAD, 1), jnp.float32),
            jax.ShapeDtypeStruct((N_PAD, 32), jnp.float32),
        ],
    )(degp, x, W1)


def _tc2(agg, dis, xw, b1, W2):
    def body(a_ref, dis_ref, xw_ref, b1_ref, w2_ref, hw2_ref):
        dis = dis_ref[...]
        h = a_ref[0] + a_ref[1] + dis * dis * xw_ref[...] + b1_ref[...]
        h = jnp.maximum(h, 0.0)
        hw2_ref[...] = jnp.dot(h, w2_ref[...],
                               preferred_element_type=jnp.float32)

    return pl.pallas_call(
        body,
        out_shape=jax.ShapeDtypeStruct((N_PAD, 16), jnp.float32),
    )(agg, dis, xw, b1, W2)


def _tc3(agg, dis, hw2, b2, W_out, b_out):
    def body(q_ref, dis_ref, hw2_ref, b2_ref, wo_ref, bo_ref, out_ref):
        dis = dis_ref[...]
        h = q_ref[0] + q_ref[1] + dis * dis * hw2_ref[...] + b2_ref[...]
        h = jnp.maximum(h, 0.0)
        out_ref[...] = jnp.dot(h, wo_ref[...],
                               preferred_element_type=jnp.float32) + bo_ref[...]

    return pl.pallas_call(
        body,
        out_shape=jax.ShapeDtypeStruct((N_PAD, 3), jnp.float32),
    )(agg, dis, hw2, b2, W_out, b_out)


def kernel(x, edge_index, edge_weight, W1, b1, W2, b2, W_out, b_out):
    ei3 = edge_index.astype(jnp.int32).reshape(2, NCHUNKS_TOT, CHUNK)
    ew2d = edge_weight.astype(jnp.float32).reshape(NCHUNKS_TOT, CHUNK)
    x_pad = jnp.concatenate(
        [x, jnp.zeros((N_PAD - N, D), jnp.float32)], axis=0)

    degp = _sc_deg(ei3, ew2d)
    dis2d, xw = _tc1(degp, x_pad, W1)
    dis = dis2d.reshape(N_PAD)

    a1 = _sc_agg(xw, dis, ei3, ew2d, 32)
    hw2 = _tc2(a1, dis2d, xw, b1.reshape(1, 32), W2)

    a2 = _sc_agg(hw2, dis, ei3, ew2d, 16)
    out = _tc3(a2, dis2d, hw2, b2.reshape(1, 16), W_out, b_out.reshape(1, 3))
    return out[:N]
